# baseline (speedup 1.0000x reference)
import jax
import jax.numpy as jnp
from jax import lax
from jax.experimental import pallas as pl
from jax.experimental.pallas import tpu as pltpu
from jax.experimental.pallas import tpu_sc as plsc

N_USERS = 25000
N_ITEMS = 25000
N = N_USERS + N_ITEMS
NP = 50176
E = 800000
D = 64
DH = 32
LAYERS = 3

NSUB = 16
EPS = E // NSUB
K = 125
INNER = 8
MIDC = EPS // (INNER * K)
RPS = NP // NSUB
RC = 112
NCH = RPS // RC

_SC_PARAMS = pltpu.CompilerParams(
    needs_layout_passes=False, use_tc_tiling_on_sc=False)


def _body_deg(src2, dst2, degs, deg_sh, idx_v, idx_v2, zv, ones_v,
              sa0, sa1, sa2, ldsem):
    c = lax.axis_index("c")
    s = lax.axis_index("s")
    z16 = jnp.zeros((16,), jnp.float32)
    nbase = s * RPS

    def _z1(i, _):
        zv[pl.ds(i * 16, 16)] = z16
        return 0
    lax.fori_loop(0, RPS // 16, _z1, 0)

    def _o1(i, _):
        ones_v[pl.ds(i * 16, 16)] = jnp.ones((16,), jnp.float32)
        return 0
    lax.fori_loop(0, 8, _o1, 0)

    pltpu.sync_copy(zv, deg_sh.at[pl.ds(nbase, RPS)])
    plsc.subcore_barrier()

    ones_k = ones_v.at[pl.ds(0, K)]
    sems = (sa0, sa1, sa2)
    for cc, arr in ((0, src2), (1, dst2)):
        @pl.when(c == cc)
        def _():
            def _ring(iv):
                handles = [None, None, None]
                for j in range(INNER):
                    b = j % 3
                    if handles[b] is not None:
                        handles[b].wait()
                    handles[b] = pltpu.async_copy(
                        ones_k, deg_sh.at[iv.at[j]], sems[b], add=True)
                for h in handles:
                    h.wait()

            pltpu.sync_copy(arr.at[s, 0], idx_v)

            def _pair(t, _):
                m0 = 2 * t
                hb = pltpu.async_copy(arr.at[s, m0 + 1], idx_v2, ldsem)
                _ring(idx_v)
                hb.wait()
                m2 = jnp.minimum(m0 + 2, MIDC - 1)
                ha = pltpu.async_copy(arr.at[s, m2], idx_v, ldsem)
                _ring(idx_v2)
                ha.wait()
                return 0
            lax.fori_loop(0, MIDC // 2, _pair, 0)
    plsc.subcore_barrier()
    pltpu.sync_copy(deg_sh.at[pl.ds(nbase, RPS)], zv)
    pltpu.sync_copy(zv, degs.at[c, pl.ds(nbase, RPS)])


BR = 1568


def _body_norm(hh_ref, do_ref, di_ref, t0_ref, in_ref, io_ref):
    on = lax.rsqrt(jnp.maximum(do_ref[...], 1.0))
    inn = lax.rsqrt(jnp.maximum(di_ref[...], 1.0))

    def expand(x):
        return jnp.concatenate(
            [jnp.broadcast_to(x[:, k:k + 1], (BR, DH)) for k in range(4)],
            axis=1)
    onx = expand(on)
    t0_ref[0] = hh_ref[0] * onx
    in_ref[...] = expand(inn)
    io_ref[...] = onx * expand(inn)


def _body_main(src2, dst2, hhalf, t0, inorm, ionorm, out, tbl,
               agg_sh, idx_s, idx_d, idx_s2, idx_d2, g0, g1, g2, agg_v,
               acc_v, n1_v, n2_v, gs0, gs1, gs2, ss0, ss1, ss2, isem, osem):
    c = lax.axis_index("c")
    s = lax.axis_index("s")
    z16 = jnp.zeros((16,), jnp.float32)
    nbase = s * RPS

    gbufs = (g0, g1, g2)
    gsems = (gs0, gs1, gs2)
    ssems = (ss0, ss1, ss2)
    NBUF = 3
    for layer in range(1, LAYERS + 1):
        last = layer == LAYERS

        if layer == 1:
            def _zb(r, _):
                n2_v[r, pl.ds(0, 16)] = z16
                n2_v[r, pl.ds(16, 16)] = z16
                return 0
            lax.fori_loop(0, RC, _zb, 0)

            def _zero_chunk(k, _):
                pltpu.sync_copy(n2_v, agg_sh.at[pl.ds(nbase + k * RC, RC)])
                return 0
            lax.fori_loop(0, NCH, _zero_chunk, 0)
        plsc.subcore_barrier()

        tcur = t0.at[c] if layer == 1 else tbl.at[c, layer - 2]

        def _ring(is_, id_):
            gh = [None] * NBUF
            sh = [None] * NBUF
            gh[0] = pltpu.async_copy(tcur.at[is_.at[0]], gbufs[0], gsems[0])
            gh[1] = pltpu.async_copy(tcur.at[is_.at[1]], gbufs[1], gsems[1])
            for j in range(INNER):
                b = j % NBUF
                gh[b].wait()
                sh[b] = pltpu.async_copy(
                    gbufs[b], agg_sh.at[id_.at[j]], ssems[b], add=True)
                nj = j + 2
                if nj < INNER:
                    nb = nj % NBUF
                    if sh[nb] is not None:
                        sh[nb].wait()
                        sh[nb] = None
                    gh[nb] = pltpu.async_copy(
                        tcur.at[is_.at[nj]], gbufs[nb], gsems[nb])
            for b in range(NBUF):
                if sh[b] is not None:
                    sh[b].wait()

        pltpu.sync_copy(src2.at[s, 0], idx_s)
        pltpu.sync_copy(dst2.at[s, 0], idx_d)

        def _edge_pair(t, _):
            m0 = 2 * t
            hb1 = pltpu.async_copy(src2.at[s, m0 + 1], idx_s2, isem)
            hb2 = pltpu.async_copy(dst2.at[s, m0 + 1], idx_d2, osem)
            _ring(idx_s, idx_d)
            hb1.wait()
            hb2.wait()
            m2 = jnp.minimum(m0 + 2, MIDC - 1)
            ha1 = pltpu.async_copy(src2.at[s, m2], idx_s, isem)
            ha2 = pltpu.async_copy(dst2.at[s, m2], idx_d, osem)
            _ring(idx_s2, idx_d2)
            ha1.wait()
            ha2.wait()
            return 0
        lax.fori_loop(0, MIDC // 2, _edge_pair, 0)
        plsc.subcore_barrier()

        if not last:
            def _zg(r, _):
                g0[r, pl.ds(0, 16)] = z16
                g0[r, pl.ds(16, 16)] = z16
                return 0
            lax.fori_loop(0, RC, _zg, 0)

        def _fin_chunk(k, _):
            rows0 = nbase + k * RC
            h1 = pltpu.async_copy(agg_sh.at[pl.ds(rows0, RC)], agg_v, isem)
            h2 = pltpu.async_copy(inorm.at[pl.ds(rows0, RC)], n1_v, osem)
            accsrc = hhalf if layer == 1 else out
            h3 = pltpu.async_copy(accsrc.at[c, pl.ds(rows0, RC)], acc_v, gs0)
            h4 = (pltpu.async_copy(ionorm.at[pl.ds(rows0, RC)], n2_v, gs1)
                  if not last else None)
            h1.wait()
            h5 = (pltpu.async_copy(
                g0.at[pl.ds(0, RC)], agg_sh.at[pl.ds(rows0, RC)], ss0)
                if not last else None)
            h2.wait()
            h3.wait()
            if h4 is not None:
                h4.wait()

            def _fin(r, _):
                for half in (0, 16):
                    sl = pl.ds(half, 16)
                    g = agg_v[r, sl]
                    a = acc_v[r, sl] + g * n1_v[r, sl]
                    if last:
                        a = a * 0.25
                    acc_v[r, sl] = a
                    if not last:
                        agg_v[r, sl] = g * n2_v[r, sl]
                return 0
            lax.fori_loop(0, RC, _fin, 0)
            o1 = pltpu.async_copy(acc_v, out.at[c, pl.ds(rows0, RC)], isem)
            o2 = (pltpu.async_copy(
                agg_v, tbl.at[c, layer - 1, pl.ds(rows0, RC)], osem)
                if not last else None)
            o1.wait()
            if o2 is not None:
                o2.wait()
            if h5 is not None:
                h5.wait()
            return 0
        lax.fori_loop(0, NCH, _fin_chunk, 0)


@jax.jit
def _lightgcn_sc(src2, dst2, hhalf):
    mesh = plsc.VectorSubcoreMesh(core_axis_name="c", subcore_axis_name="s")
    f32 = jnp.float32

    deg_run = pl.kernel(
        _body_deg,
        mesh=mesh,
        compiler_params=_SC_PARAMS,
        out_type=[jax.ShapeDtypeStruct((2, NP), f32)],
        scratch_types=[
            pltpu.VMEM_SHARED((NP,), f32),
            pltpu.VMEM((INNER, K), jnp.int32),
            pltpu.VMEM((INNER, K), jnp.int32),
            pltpu.VMEM((RPS,), f32),
            pltpu.VMEM((128,), f32),
        ] + [pltpu.SemaphoreType.DMA] * 4,
    )
    [degs] = deg_run(src2, dst2)

    t0, inorm, ionorm = pl.pallas_call(
        _body_norm,
        grid=(2, NP // 4 // BR),
        in_specs=[
            pl.BlockSpec((1, BR, 128), lambda cb, i: (cb, i, 0)),
            pl.BlockSpec((BR, 4), lambda cb, i: (i, 0)),
            pl.BlockSpec((BR, 4), lambda cb, i: (i, 0)),
        ],
        out_specs=[
            pl.BlockSpec((1, BR, 128), lambda cb, i: (cb, i, 0)),
            pl.BlockSpec((BR, 128), lambda cb, i: (i, 0)),
            pl.BlockSpec((BR, 128), lambda cb, i: (i, 0)),
        ],
        out_shape=[
            jax.ShapeDtypeStruct((2, NP // 4, 128), f32),
            jax.ShapeDtypeStruct((NP // 4, 128), f32),
            jax.ShapeDtypeStruct((NP // 4, 128), f32),
        ],
    )(hhalf.reshape(2, NP // 4, 128),
      degs[0].reshape(NP // 4, 4), degs[1].reshape(NP // 4, 4))
    t0 = t0.reshape(2, NP, DH)
    inorm = inorm.reshape(NP, DH)
    ionorm = ionorm.reshape(NP, DH)

    main_run = pl.kernel(
        _body_main,
        mesh=mesh,
        compiler_params=_SC_PARAMS,
        out_type=[
            jax.ShapeDtypeStruct((2, NP, DH), f32),
            jax.ShapeDtypeStruct((2, 2, NP, DH), f32),
        ],
        scratch_types=[
            pltpu.VMEM_SHARED((NP, DH), f32),
            pltpu.VMEM((INNER, K), jnp.int32),
            pltpu.VMEM((INNER, K), jnp.int32),
            pltpu.VMEM((INNER, K), jnp.int32),
            pltpu.VMEM((INNER, K), jnp.int32),
            pltpu.VMEM((K, DH), f32),
            pltpu.VMEM((K, DH), f32),
            pltpu.VMEM((K, DH), f32),
            pltpu.VMEM((RC, DH), f32),
            pltpu.VMEM((RC, DH), f32),
            pltpu.VMEM((RC, DH), f32),
            pltpu.VMEM((RC, DH), f32),
        ] + [pltpu.SemaphoreType.DMA] * 8,
    )
    out, _ = main_run(src2, dst2, hhalf, t0, inorm, ionorm)
    return out


def kernel(user_emb, item_emb, edge_index):
    hcat = jnp.concatenate([user_emb, item_emb], axis=0)
    hpad = jnp.zeros((NP, D), jnp.float32).at[:N].set(hcat)
    hhalf = jnp.stack([hpad[:, :DH], hpad[:, DH:]])
    src2 = edge_index[0].reshape(NSUB, MIDC, INNER, K)
    dst2 = edge_index[1].reshape(NSUB, MIDC, INNER, K)
    out = _lightgcn_sc(src2, dst2, hhalf)
    full = jnp.concatenate([out[0, :N], out[1, :N]], axis=1)
    return full[:N_USERS], full[N_USERS:]


# speedup vs baseline: 1.0198x; 1.0198x over previous
import jax
import jax.numpy as jnp
from jax import lax
from jax.experimental import pallas as pl
from jax.experimental.pallas import tpu as pltpu
from jax.experimental.pallas import tpu_sc as plsc

N_USERS = 25000
N_ITEMS = 25000
N = N_USERS + N_ITEMS
NP = 50176
E = 800000
D = 64
DH = 32
LAYERS = 3

NSUB = 16
EPS = E // NSUB
K = 125
INNER = 8
MIDC = EPS // (INNER * K)
RPS = NP // NSUB
RC = 56
NCH = RPS // RC

_SC_PARAMS = pltpu.CompilerParams(
    needs_layout_passes=False, use_tc_tiling_on_sc=False)


def _body_deg(src2, dst2, degs, deg_sh, idx_v, idx_v2, zv, ones_v,
              sa0, sa1, sa2, ldsem):
    c = lax.axis_index("c")
    s = lax.axis_index("s")
    z16 = jnp.zeros((16,), jnp.float32)
    nbase = s * RPS

    def _z1(i, _):
        zv[pl.ds(i * 16, 16)] = z16
        return 0
    lax.fori_loop(0, RPS // 16, _z1, 0)

    def _o1(i, _):
        ones_v[pl.ds(i * 16, 16)] = jnp.ones((16,), jnp.float32)
        return 0
    lax.fori_loop(0, 8, _o1, 0)

    pltpu.sync_copy(zv, deg_sh.at[pl.ds(nbase, RPS)])
    plsc.subcore_barrier()

    ones_k = ones_v.at[pl.ds(0, K)]
    sems = (sa0, sa1, sa2)
    for cc, arr in ((0, src2), (1, dst2)):
        @pl.when(c == cc)
        def _():
            def _ring(iv):
                handles = [None, None, None]
                for j in range(INNER):
                    b = j % 3
                    if handles[b] is not None:
                        handles[b].wait()
                    handles[b] = pltpu.async_copy(
                        ones_k, deg_sh.at[iv.at[j]], sems[b], add=True)
                for h in handles:
                    h.wait()

            pltpu.sync_copy(arr.at[s, 0], idx_v)

            def _pair(t, _):
                m0 = 2 * t
                hb = pltpu.async_copy(arr.at[s, m0 + 1], idx_v2, ldsem)
                _ring(idx_v)
                hb.wait()
                m2 = jnp.minimum(m0 + 2, MIDC - 1)
                ha = pltpu.async_copy(arr.at[s, m2], idx_v, ldsem)
                _ring(idx_v2)
                ha.wait()
                return 0
            lax.fori_loop(0, MIDC // 2, _pair, 0)
    plsc.subcore_barrier()
    pltpu.sync_copy(deg_sh.at[pl.ds(nbase, RPS)], zv)
    pltpu.sync_copy(zv, degs.at[c, pl.ds(nbase, RPS)])


BR = 1568


def _body_norm(hh_ref, do_ref, di_ref, t0_ref, in_ref, io_ref):
    on = lax.rsqrt(jnp.maximum(do_ref[...], 1.0))
    inn = lax.rsqrt(jnp.maximum(di_ref[...], 1.0))

    def expand(x):
        return jnp.concatenate(
            [jnp.broadcast_to(x[:, k:k + 1], (BR, DH)) for k in range(4)],
            axis=1)
    onx = expand(on)
    t0_ref[0] = hh_ref[0] * onx
    in_ref[...] = expand(inn)
    io_ref[...] = onx * expand(inn)


def _body_main(src2, dst2, hhalf, t0, inorm, ionorm, out, tbl,
               agg_sh, idx_s, idx_d, idx_s2, idx_d2, g0, g1, g2, g3, agg_v,
               acc_v, n1_v, n2_v, gs0, gs1, gs2, gs3, ss0, ss1, ss2, ss3,
               isem, osem):
    c = lax.axis_index("c")
    s = lax.axis_index("s")
    z16 = jnp.zeros((16,), jnp.float32)
    nbase = s * RPS

    gbufs = (g0, g1, g2, g3)
    gsems = (gs0, gs1, gs2, gs3)
    ssems = (ss0, ss1, ss2, ss3)
    NBUF = 4
    for layer in range(1, LAYERS + 1):
        last = layer == LAYERS

        if layer == 1:
            def _zb(r, _):
                n2_v[r, pl.ds(0, 16)] = z16
                n2_v[r, pl.ds(16, 16)] = z16
                return 0
            lax.fori_loop(0, RC, _zb, 0)

            def _zero_chunk(k, _):
                pltpu.sync_copy(n2_v, agg_sh.at[pl.ds(nbase + k * RC, RC)])
                return 0
            lax.fori_loop(0, NCH, _zero_chunk, 0)
        plsc.subcore_barrier()

        tcur = t0.at[c] if layer == 1 else tbl.at[c, layer - 2]

        def _ring(is_, id_):
            gh = [None] * NBUF
            sh = [None] * NBUF
            for p in range(NBUF - 1):
                gh[p] = pltpu.async_copy(
                    tcur.at[is_.at[p]], gbufs[p], gsems[p])
            for j in range(INNER):
                b = j % NBUF
                gh[b].wait()
                sh[b] = pltpu.async_copy(
                    gbufs[b], agg_sh.at[id_.at[j]], ssems[b], add=True)
                nj = j + NBUF - 1
                if nj < INNER:
                    nb = nj % NBUF
                    if sh[nb] is not None:
                        sh[nb].wait()
                        sh[nb] = None
                    gh[nb] = pltpu.async_copy(
                        tcur.at[is_.at[nj]], gbufs[nb], gsems[nb])
            for b in range(NBUF):
                if sh[b] is not None:
                    sh[b].wait()

        pltpu.sync_copy(src2.at[s, 0], idx_s)
        pltpu.sync_copy(dst2.at[s, 0], idx_d)

        def _edge_pair(t, _):
            m0 = 2 * t
            hb1 = pltpu.async_copy(src2.at[s, m0 + 1], idx_s2, isem)
            hb2 = pltpu.async_copy(dst2.at[s, m0 + 1], idx_d2, osem)
            _ring(idx_s, idx_d)
            hb1.wait()
            hb2.wait()
            m2 = jnp.minimum(m0 + 2, MIDC - 1)
            ha1 = pltpu.async_copy(src2.at[s, m2], idx_s, isem)
            ha2 = pltpu.async_copy(dst2.at[s, m2], idx_d, osem)
            _ring(idx_s2, idx_d2)
            ha1.wait()
            ha2.wait()
            return 0
        lax.fori_loop(0, MIDC // 2, _edge_pair, 0)
        plsc.subcore_barrier()

        if not last:
            def _zg(r, _):
                g0[r, pl.ds(0, 16)] = z16
                g0[r, pl.ds(16, 16)] = z16
                return 0
            lax.fori_loop(0, RC, _zg, 0)

        def _fin_chunk(k, _):
            rows0 = nbase + k * RC
            h1 = pltpu.async_copy(agg_sh.at[pl.ds(rows0, RC)], agg_v, isem)
            h2 = pltpu.async_copy(inorm.at[pl.ds(rows0, RC)], n1_v, osem)
            accsrc = hhalf if layer == 1 else out
            h3 = pltpu.async_copy(accsrc.at[c, pl.ds(rows0, RC)], acc_v, gs0)
            h4 = (pltpu.async_copy(ionorm.at[pl.ds(rows0, RC)], n2_v, gs1)
                  if not last else None)
            h1.wait()
            h5 = (pltpu.async_copy(
                g0.at[pl.ds(0, RC)], agg_sh.at[pl.ds(rows0, RC)], ss0)
                if not last else None)
            h2.wait()
            h3.wait()
            if h4 is not None:
                h4.wait()

            def _fin(r, _):
                for half in (0, 16):
                    sl = pl.ds(half, 16)
                    g = agg_v[r, sl]
                    a = acc_v[r, sl] + g * n1_v[r, sl]
                    if last:
                        a = a * 0.25
                    acc_v[r, sl] = a
                    if not last:
                        agg_v[r, sl] = g * n2_v[r, sl]
                return 0
            lax.fori_loop(0, RC, _fin, 0)
            o1 = pltpu.async_copy(acc_v, out.at[c, pl.ds(rows0, RC)], isem)
            o2 = (pltpu.async_copy(
                agg_v, tbl.at[c, layer - 1, pl.ds(rows0, RC)], osem)
                if not last else None)
            o1.wait()
            if o2 is not None:
                o2.wait()
            if h5 is not None:
                h5.wait()
            return 0
        lax.fori_loop(0, NCH, _fin_chunk, 0)


@jax.jit
def _lightgcn_sc(src2, dst2, hhalf):
    mesh = plsc.VectorSubcoreMesh(core_axis_name="c", subcore_axis_name="s")
    f32 = jnp.float32

    deg_run = pl.kernel(
        _body_deg,
        mesh=mesh,
        compiler_params=_SC_PARAMS,
        out_type=[jax.ShapeDtypeStruct((2, NP), f32)],
        scratch_types=[
            pltpu.VMEM_SHARED((NP,), f32),
            pltpu.VMEM((INNER, K), jnp.int32),
            pltpu.VMEM((INNER, K), jnp.int32),
            pltpu.VMEM((RPS,), f32),
            pltpu.VMEM((128,), f32),
        ] + [pltpu.SemaphoreType.DMA] * 4,
    )
    [degs] = deg_run(src2, dst2)

    t0, inorm, ionorm = pl.pallas_call(
        _body_norm,
        grid=(2, NP // 4 // BR),
        in_specs=[
            pl.BlockSpec((1, BR, 128), lambda cb, i: (cb, i, 0)),
            pl.BlockSpec((BR, 4), lambda cb, i: (i, 0)),
            pl.BlockSpec((BR, 4), lambda cb, i: (i, 0)),
        ],
        out_specs=[
            pl.BlockSpec((1, BR, 128), lambda cb, i: (cb, i, 0)),
            pl.BlockSpec((BR, 128), lambda cb, i: (i, 0)),
            pl.BlockSpec((BR, 128), lambda cb, i: (i, 0)),
        ],
        out_shape=[
            jax.ShapeDtypeStruct((2, NP // 4, 128), f32),
            jax.ShapeDtypeStruct((NP // 4, 128), f32),
            jax.ShapeDtypeStruct((NP // 4, 128), f32),
        ],
    )(hhalf.reshape(2, NP // 4, 128),
      degs[0].reshape(NP // 4, 4), degs[1].reshape(NP // 4, 4))
    t0 = t0.reshape(2, NP, DH)
    inorm = inorm.reshape(NP, DH)
    ionorm = ionorm.reshape(NP, DH)

    main_run = pl.kernel(
        _body_main,
        mesh=mesh,
        compiler_params=_SC_PARAMS,
        out_type=[
            jax.ShapeDtypeStruct((2, NP, DH), f32),
            jax.ShapeDtypeStruct((2, 2, NP, DH), f32),
        ],
        scratch_types=[
            pltpu.VMEM_SHARED((NP, DH), f32),
            pltpu.VMEM((INNER, K), jnp.int32),
            pltpu.VMEM((INNER, K), jnp.int32),
            pltpu.VMEM((INNER, K), jnp.int32),
            pltpu.VMEM((INNER, K), jnp.int32),
            pltpu.VMEM((K, DH), f32),
            pltpu.VMEM((K, DH), f32),
            pltpu.VMEM((K, DH), f32),
            pltpu.VMEM((K, DH), f32),
            pltpu.VMEM((RC, DH), f32),
            pltpu.VMEM((RC, DH), f32),
            pltpu.VMEM((RC, DH), f32),
            pltpu.VMEM((RC, DH), f32),
        ] + [pltpu.SemaphoreType.DMA] * 10,
    )
    out, _ = main_run(src2, dst2, hhalf, t0, inorm, ionorm)
    return out


def kernel(user_emb, item_emb, edge_index):
    hcat = jnp.concatenate([user_emb, item_emb], axis=0)
    hpad = jnp.zeros((NP, D), jnp.float32).at[:N].set(hcat)
    hhalf = jnp.stack([hpad[:, :DH], hpad[:, DH:]])
    src2 = edge_index[0].reshape(NSUB, MIDC, INNER, K)
    dst2 = edge_index[1].reshape(NSUB, MIDC, INNER, K)
    out = _lightgcn_sc(src2, dst2, hhalf)
    full = jnp.concatenate([out[0, :N], out[1, :N]], axis=1)
    return full[:N_USERS], full[N_USERS:]
